# baseline (speedup 1.0000x reference)
import jax
import jax.numpy as jnp
from jax.experimental import pallas as pl

N_TOKENS = 8192
D_MODEL = 2048
NUM_EXPERTS = 64
BLOCK_T = 512


def _router_body(x_ref, cs_ref, wt_ref, wg_ref, bg_ref,
                 gates_ref, idx_ref, probs_ref):
    xb = x_ref[...]
    logits = jnp.dot(xb, wt_ref[...],
                     preferred_element_type=jnp.float32)
    bias = cs_ref[...] * wg_ref[...] + bg_ref[...]
    logits = logits + bias
    m = jnp.max(logits, axis=-1, keepdims=True)
    e = jnp.exp(logits - m)
    s = jnp.sum(e, axis=-1, keepdims=True)
    probs_ref[...] = e / s
    gates_ref[...] = 1.0 / s
    iota = jax.lax.broadcasted_iota(jnp.int32, logits.shape, 1)
    idx_ref[...] = jnp.min(
        jnp.where(logits == m, iota, NUM_EXPERTS), axis=-1, keepdims=True)


def kernel(x, complexity_signal, W_router, W_gate, b_gate):
    wt = W_router.T
    cs = complexity_signal[:, None]
    wg = W_gate.T
    bg = b_gate[None, :]
    n_blocks = N_TOKENS // BLOCK_T
    gates2d, idx2d, probs = pl.pallas_call(
        _router_body,
        grid=(n_blocks,),
        in_specs=[
            pl.BlockSpec((BLOCK_T, D_MODEL), lambda i: (i, 0)),
            pl.BlockSpec((BLOCK_T, 1), lambda i: (i, 0)),
            pl.BlockSpec((D_MODEL, NUM_EXPERTS), lambda i: (0, 0)),
            pl.BlockSpec((1, NUM_EXPERTS), lambda i: (0, 0)),
            pl.BlockSpec((1, NUM_EXPERTS), lambda i: (0, 0)),
        ],
        out_specs=[
            pl.BlockSpec((BLOCK_T, 1), lambda i: (i, 0)),
            pl.BlockSpec((BLOCK_T, 1), lambda i: (i, 0)),
            pl.BlockSpec((BLOCK_T, NUM_EXPERTS), lambda i: (i, 0)),
        ],
        out_shape=[
            jax.ShapeDtypeStruct((N_TOKENS, 1), jnp.float32),
            jax.ShapeDtypeStruct((N_TOKENS, 1), jnp.int32),
            jax.ShapeDtypeStruct((N_TOKENS, NUM_EXPERTS), jnp.float32),
        ],
    )(x, cs, wt, wg, bg)
    return gates2d[:, 0], idx2d[:, 0], probs


# speedup vs baseline: 1.0803x; 1.0803x over previous
import jax
import jax.numpy as jnp
from jax.experimental import pallas as pl

N_TOKENS = 8192
D_MODEL = 2048
NUM_EXPERTS = 64
BLOCK_T = 1024


def _router_body(x_ref, cs_ref, wt_ref, wg_ref, bg_ref,
                 gates_ref, idx_ref, probs_ref):
    xb = x_ref[...]
    logits = jnp.dot(xb, wt_ref[...],
                     preferred_element_type=jnp.float32)
    bias = cs_ref[...] * wg_ref[...] + bg_ref[...]
    logits = logits + bias
    m = jnp.max(logits, axis=-1, keepdims=True)
    e = jnp.exp(logits - m)
    s = jnp.sum(e, axis=-1, keepdims=True)
    probs_ref[...] = e / s
    gates_ref[...] = 1.0 / s
    iota = jax.lax.broadcasted_iota(jnp.int32, logits.shape, 1)
    idx_ref[...] = jnp.min(
        jnp.where(logits == m, iota, NUM_EXPERTS), axis=-1, keepdims=True)


def kernel(x, complexity_signal, W_router, W_gate, b_gate):
    wt = W_router.T
    cs = complexity_signal[:, None]
    wg = W_gate.T
    bg = b_gate[None, :]
    n_blocks = N_TOKENS // BLOCK_T
    gates2d, idx2d, probs = pl.pallas_call(
        _router_body,
        grid=(n_blocks,),
        in_specs=[
            pl.BlockSpec((BLOCK_T, D_MODEL), lambda i: (i, 0)),
            pl.BlockSpec((BLOCK_T, 1), lambda i: (i, 0)),
            pl.BlockSpec((D_MODEL, NUM_EXPERTS), lambda i: (0, 0)),
            pl.BlockSpec((1, NUM_EXPERTS), lambda i: (0, 0)),
            pl.BlockSpec((1, NUM_EXPERTS), lambda i: (0, 0)),
        ],
        out_specs=[
            pl.BlockSpec((BLOCK_T, 1), lambda i: (i, 0)),
            pl.BlockSpec((BLOCK_T, 1), lambda i: (i, 0)),
            pl.BlockSpec((BLOCK_T, NUM_EXPERTS), lambda i: (i, 0)),
        ],
        out_shape=[
            jax.ShapeDtypeStruct((N_TOKENS, 1), jnp.float32),
            jax.ShapeDtypeStruct((N_TOKENS, 1), jnp.int32),
            jax.ShapeDtypeStruct((N_TOKENS, NUM_EXPERTS), jnp.float32),
        ],
    )(x, cs, wt, wg, bg)
    return gates2d[:, 0], idx2d[:, 0], probs


# speedup vs baseline: 1.0940x; 1.0127x over previous
import jax
import jax.numpy as jnp
from jax.experimental import pallas as pl
from jax.experimental.pallas import tpu as pltpu

N_TOKENS = 8192
D_MODEL = 2048
NUM_EXPERTS = 64
BLOCK_T = 1024


def _router_body(x_ref, cs_ref, wt_ref, wg_ref, bg_ref,
                 gates_ref, idx_ref, probs_ref):
    xb = x_ref[...]
    logits = jnp.dot(xb, wt_ref[...],
                     preferred_element_type=jnp.float32)
    bias = cs_ref[...] * wg_ref[...] + bg_ref[...]
    logits = logits + bias
    m = jnp.max(logits, axis=-1, keepdims=True)
    e = jnp.exp(logits - m)
    s = jnp.sum(e, axis=-1, keepdims=True)
    probs_ref[...] = e / s
    gates_ref[...] = 1.0 / s
    iota = jax.lax.broadcasted_iota(jnp.int32, logits.shape, 1)
    idx_ref[...] = jnp.min(
        jnp.where(logits == m, iota, NUM_EXPERTS), axis=-1, keepdims=True)


def kernel(x, complexity_signal, W_router, W_gate, b_gate):
    wt = W_router.T
    cs = complexity_signal[:, None]
    wg = W_gate.T
    bg = b_gate[None, :]
    n_blocks = N_TOKENS // BLOCK_T
    gates2d, idx2d, probs = pl.pallas_call(
        _router_body,
        grid=(n_blocks,),
        in_specs=[
            pl.BlockSpec((BLOCK_T, D_MODEL), lambda i: (i, 0)),
            pl.BlockSpec((BLOCK_T, 1), lambda i: (i, 0)),
            pl.BlockSpec((D_MODEL, NUM_EXPERTS), lambda i: (0, 0)),
            pl.BlockSpec((1, NUM_EXPERTS), lambda i: (0, 0)),
            pl.BlockSpec((1, NUM_EXPERTS), lambda i: (0, 0)),
        ],
        out_specs=[
            pl.BlockSpec((BLOCK_T, 1), lambda i: (i, 0)),
            pl.BlockSpec((BLOCK_T, 1), lambda i: (i, 0)),
            pl.BlockSpec((BLOCK_T, NUM_EXPERTS), lambda i: (i, 0)),
        ],
        out_shape=[
            jax.ShapeDtypeStruct((N_TOKENS, 1), jnp.float32),
            jax.ShapeDtypeStruct((N_TOKENS, 1), jnp.int32),
            jax.ShapeDtypeStruct((N_TOKENS, NUM_EXPERTS), jnp.float32),
        ],
        compiler_params=pltpu.CompilerParams(
            dimension_semantics=("parallel",)),
    )(x, cs, wt, wg, bg)
    return gates2d[:, 0], idx2d[:, 0], probs
